# Initial kernel scaffold; baseline (speedup 1.0000x reference)
import jax
import jax.numpy as jnp
from jax.experimental import pallas as pl


def kernel(x, edge_index, h0, c0, W1, b1, W2, b2, W_ih, W_hh, b_ih, b_hh, W_fc, b_fc):
    raise NotImplementedError("write your pallas kernel here")


# speedup vs baseline: 5.3807x; 5.3807x over previous
import functools

import jax
import jax.numpy as jnp
from jax import lax
from jax.experimental import pallas as pl
from jax.experimental.pallas import tpu as pltpu

N = 10000
E = 320000
IN = 128
H = 256
LH = 256
OUT = 128


def _mm_scale_body(a_ref, w_ref, dinv_ref, o_ref):
    acc = jnp.dot(a_ref[...], w_ref[...], preferred_element_type=jnp.float32)
    o_ref[...] = acc * dinv_ref[...]


def _mm_scale(a, w, dinv, blk=2000):
    n, kin = a.shape
    kout = w.shape[1]
    grid = n // blk
    return pl.pallas_call(
        _mm_scale_body,
        grid=(grid,),
        in_specs=[
            pl.BlockSpec((blk, kin), lambda i: (i, 0)),
            pl.BlockSpec((kin, kout), lambda i: (0, 0)),
            pl.BlockSpec((blk, 1), lambda i: (i, 0)),
        ],
        out_specs=pl.BlockSpec((blk, kout), lambda i: (i, 0)),
        out_shape=jax.ShapeDtypeStruct((n, kout), jnp.float32),
    )(a, w, dinv)


def _cmb_mm_body(scale_out, aggA_ref, aggB_ref, xs_ref, dinv_ref, b_ref,
                 w_ref, b2_ref, o_ref):
    h = (aggA_ref[...] + aggB_ref[...] + xs_ref[...]) * dinv_ref[...] + b_ref[...]
    h = jnp.maximum(h, 0.0)
    acc = jnp.dot(h, w_ref[...], preferred_element_type=jnp.float32)
    if scale_out:
        o_ref[...] = acc * dinv_ref[...]
    else:
        o_ref[...] = acc + b2_ref[...]


def _cmb_mm(aggA, aggB, xs, dinv, b, w, b2, scale_out, blk=1000):
    n, kin = xs.shape
    kout = w.shape[1]
    grid = n // blk
    return pl.pallas_call(
        functools.partial(_cmb_mm_body, scale_out),
        grid=(grid,),
        in_specs=[
            pl.BlockSpec((blk, kin), lambda i: (i, 0)),
            pl.BlockSpec((blk, kin), lambda i: (i, 0)),
            pl.BlockSpec((blk, kin), lambda i: (i, 0)),
            pl.BlockSpec((blk, 1), lambda i: (i, 0)),
            pl.BlockSpec((1, kin), lambda i: (0, 0)),
            pl.BlockSpec((kin, kout), lambda i: (0, 0)),
            pl.BlockSpec((1, kout), lambda i: (0, 0)),
        ],
        out_specs=pl.BlockSpec((blk, kout), lambda i: (i, 0)),
        out_shape=jax.ShapeDtypeStruct((n, kout), jnp.float32),
    )(aggA, aggB, xs, dinv, b, w, b2)


def _lstm_body(tb, xi_ref, h0_ref, c0_ref, whh_ref, bhh_ref, wfc_ref, bfc_ref,
               logp_ref, hT_ref, cT_ref, h_s, c_s, hs_s):
    @pl.when(pl.program_id(0) == 0)
    def _init():
        h_s[...] = h0_ref[...]
        c_s[...] = c0_ref[...]

    def step(t, carry):
        h, c = carry
        g = xi_ref[pl.ds(t, 1), :] + jnp.dot(
            h, whh_ref[...], preferred_element_type=jnp.float32) + bhh_ref[...]
        i = jax.nn.sigmoid(g[:, 0:LH])
        f = jax.nn.sigmoid(g[:, LH:2 * LH])
        gg = jnp.tanh(g[:, 2 * LH:3 * LH])
        o = jax.nn.sigmoid(g[:, 3 * LH:4 * LH])
        c = f * c + i * gg
        h = o * jnp.tanh(c)
        hs_s[pl.ds(t, 1), :] = h
        return (h, c)

    h, c = lax.fori_loop(0, tb, step, (h_s[...], c_s[...]))
    h_s[...] = h
    c_s[...] = c
    out = jnp.dot(hs_s[...], wfc_ref[...], preferred_element_type=jnp.float32) \
        + bfc_ref[...]
    m = jnp.max(out, axis=-1, keepdims=True)
    lse = jnp.log(jnp.sum(jnp.exp(out - m), axis=-1, keepdims=True))
    logp_ref[...] = out - m - lse
    hT_ref[...] = h
    cT_ref[...] = c


def _lstm_head(xi, h0, c0, whh_t, bhh, wfc, bfc, tb=400):
    n = xi.shape[0]
    grid = n // tb
    return pl.pallas_call(
        functools.partial(_lstm_body, tb),
        grid=(grid,),
        in_specs=[
            pl.BlockSpec((tb, 4 * LH), lambda i: (i, 0)),
            pl.BlockSpec((1, LH), lambda i: (0, 0)),
            pl.BlockSpec((1, LH), lambda i: (0, 0)),
            pl.BlockSpec((LH, 4 * LH), lambda i: (0, 0)),
            pl.BlockSpec((1, 4 * LH), lambda i: (0, 0)),
            pl.BlockSpec((LH, OUT), lambda i: (0, 0)),
            pl.BlockSpec((1, OUT), lambda i: (0, 0)),
        ],
        out_specs=[
            pl.BlockSpec((tb, OUT), lambda i: (i, 0)),
            pl.BlockSpec((1, LH), lambda i: (0, 0)),
            pl.BlockSpec((1, LH), lambda i: (0, 0)),
        ],
        out_shape=[
            jax.ShapeDtypeStruct((n, OUT), jnp.float32),
            jax.ShapeDtypeStruct((1, LH), jnp.float32),
            jax.ShapeDtypeStruct((1, LH), jnp.float32),
        ],
        scratch_shapes=[
            pltpu.VMEM((1, LH), jnp.float32),
            pltpu.VMEM((1, LH), jnp.float32),
            pltpu.VMEM((tb, LH), jnp.float32),
        ],
    )(xi, h0, c0, whh_t, bhh, wfc, bfc)


def _dinv_body(d_ref, o_ref):
    o_ref[...] = lax.rsqrt(d_ref[0] + d_ref[1] + 1.0)


def _dinv(degp, blk=2000):
    n = degp.shape[1]
    return pl.pallas_call(
        _dinv_body,
        grid=(n // blk,),
        in_specs=[pl.BlockSpec((2, blk, 1), lambda i: (0, i, 0))],
        out_specs=pl.BlockSpec((blk, 1), lambda i: (i, 0)),
        out_shape=jax.ShapeDtypeStruct((n, 1), jnp.float32),
    )(degp)


def _deg_partials(dst):
    deg = jax.ops.segment_sum(jnp.ones((E,), jnp.float32), dst, num_segments=N)
    z = jnp.zeros((N,), jnp.float32)
    return jnp.stack([deg, z])[:, :, None]


def _agg_partials(xs, src, dst):
    agg = jax.ops.segment_sum(xs[src], dst, num_segments=N)
    return agg, jnp.zeros_like(agg)


def kernel(x, edge_index, h0, c0, W1, b1, W2, b2, W_ih, W_hh, b_ih, b_hh,
           W_fc, b_fc):
    src = edge_index[0]
    dst = edge_index[1]

    degp = _deg_partials(dst)
    dinv = _dinv(degp)

    xs1 = _mm_scale(x, W1, dinv)
    a1A, a1B = _agg_partials(xs1, src, dst)
    xs2 = _cmb_mm(a1A, a1B, xs1, dinv, b1[None, :], W2, b1[None, :],
                  scale_out=True)
    a2A, a2B = _agg_partials(xs2, src, dst)
    xi = _cmb_mm(a2A, a2B, xs2, dinv, b2[None, :], W_ih.T,
                 b_ih[None, :], scale_out=False)

    logp, hT, cT = _lstm_head(xi, h0[0], c0[0], W_hh.T, b_hh[None, :],
                              W_fc, b_fc[None, :])
    return logp[None, :, :], hT[None, :, :], cT[None, :, :]


# speedup vs baseline: 10.0535x; 1.8684x over previous
import functools

import jax
import jax.numpy as jnp
from jax import lax
from jax.experimental import pallas as pl
from jax.experimental.pallas import tpu as pltpu
from jax.experimental.pallas import tpu_sc as plsc

N = 10000
E = 320000
IN = 128
H = 256
LH = 256
OUT = 128


def _mm_scale_body(a_ref, w_ref, dinv_ref, o_ref):
    acc = jnp.dot(a_ref[...], w_ref[...], preferred_element_type=jnp.float32)
    o_ref[...] = acc * dinv_ref[...]


def _mm_scale(a, w, dinv, blk=2000):
    n, kin = a.shape
    kout = w.shape[1]
    grid = n // blk
    return pl.pallas_call(
        _mm_scale_body,
        grid=(grid,),
        in_specs=[
            pl.BlockSpec((blk, kin), lambda i: (i, 0)),
            pl.BlockSpec((kin, kout), lambda i: (0, 0)),
            pl.BlockSpec((blk, 1), lambda i: (i, 0)),
        ],
        out_specs=pl.BlockSpec((blk, kout), lambda i: (i, 0)),
        out_shape=jax.ShapeDtypeStruct((n, kout), jnp.float32),
    )(a, w, dinv)


def _cmb_mm_body(scale_out, aggA_ref, aggB_ref, xs_ref, dinv_ref, b_ref,
                 w_ref, b2_ref, o_ref):
    h = (aggA_ref[...] + aggB_ref[...] + xs_ref[...]) * dinv_ref[...] + b_ref[...]
    h = jnp.maximum(h, 0.0)
    acc = jnp.dot(h, w_ref[...], preferred_element_type=jnp.float32)
    if scale_out:
        o_ref[...] = acc * dinv_ref[...]
    else:
        o_ref[...] = acc + b2_ref[...]


def _cmb_mm(aggA, aggB, xs, dinv, b, w, b2, scale_out, blk=1000):
    n, kin = xs.shape
    kout = w.shape[1]
    grid = n // blk
    return pl.pallas_call(
        functools.partial(_cmb_mm_body, scale_out),
        grid=(grid,),
        in_specs=[
            pl.BlockSpec((blk, kin), lambda i: (i, 0)),
            pl.BlockSpec((blk, kin), lambda i: (i, 0)),
            pl.BlockSpec((blk, kin), lambda i: (i, 0)),
            pl.BlockSpec((blk, 1), lambda i: (i, 0)),
            pl.BlockSpec((1, kin), lambda i: (0, 0)),
            pl.BlockSpec((kin, kout), lambda i: (0, 0)),
            pl.BlockSpec((1, kout), lambda i: (0, 0)),
        ],
        out_specs=pl.BlockSpec((blk, kout), lambda i: (i, 0)),
        out_shape=jax.ShapeDtypeStruct((n, kout), jnp.float32),
    )(aggA, aggB, xs, dinv, b, w, b2)


def _lstm_body(tb, xi_ref, h0_ref, c0_ref, whh_ref, bhh_ref, wfc_ref, bfc_ref,
               logp_ref, hT_ref, cT_ref, h_s, c_s, hs_s):
    @pl.when(pl.program_id(0) == 0)
    def _init():
        h_s[...] = h0_ref[...]
        c_s[...] = c0_ref[...]

    def step(t, carry):
        h, c = carry
        g = xi_ref[pl.ds(t, 1), :] + jnp.dot(
            h, whh_ref[...], preferred_element_type=jnp.float32) + bhh_ref[...]
        i = jax.nn.sigmoid(g[:, 0:LH])
        f = jax.nn.sigmoid(g[:, LH:2 * LH])
        gg = jnp.tanh(g[:, 2 * LH:3 * LH])
        o = jax.nn.sigmoid(g[:, 3 * LH:4 * LH])
        c = f * c + i * gg
        h = o * jnp.tanh(c)
        hs_s[pl.ds(t, 1), :] = h
        return (h, c)

    h, c = lax.fori_loop(0, tb, step, (h_s[...], c_s[...]))
    h_s[...] = h
    c_s[...] = c
    out = jnp.dot(hs_s[...], wfc_ref[...], preferred_element_type=jnp.float32) \
        + bfc_ref[...]
    m = jnp.max(out, axis=-1, keepdims=True)
    lse = jnp.log(jnp.sum(jnp.exp(out - m), axis=-1, keepdims=True))
    logp_ref[...] = out - m - lse
    hT_ref[...] = h
    cT_ref[...] = c


def _lstm_head(xi, h0, c0, whh_t, bhh, wfc, bfc, tb=400):
    n = xi.shape[0]
    grid = n // tb
    return pl.pallas_call(
        functools.partial(_lstm_body, tb),
        grid=(grid,),
        in_specs=[
            pl.BlockSpec((tb, 4 * LH), lambda i: (i, 0)),
            pl.BlockSpec((1, LH), lambda i: (0, 0)),
            pl.BlockSpec((1, LH), lambda i: (0, 0)),
            pl.BlockSpec((LH, 4 * LH), lambda i: (0, 0)),
            pl.BlockSpec((1, 4 * LH), lambda i: (0, 0)),
            pl.BlockSpec((LH, OUT), lambda i: (0, 0)),
            pl.BlockSpec((1, OUT), lambda i: (0, 0)),
        ],
        out_specs=[
            pl.BlockSpec((tb, OUT), lambda i: (i, 0)),
            pl.BlockSpec((1, LH), lambda i: (0, 0)),
            pl.BlockSpec((1, LH), lambda i: (0, 0)),
        ],
        out_shape=[
            jax.ShapeDtypeStruct((n, OUT), jnp.float32),
            jax.ShapeDtypeStruct((1, LH), jnp.float32),
            jax.ShapeDtypeStruct((1, LH), jnp.float32),
        ],
        scratch_shapes=[
            pltpu.VMEM((1, LH), jnp.float32),
            pltpu.VMEM((1, LH), jnp.float32),
            pltpu.VMEM((tb, LH), jnp.float32),
        ],
    )(xi, h0, c0, whh_t, bhh, wfc, bfc)


def _dinv_body(d_ref, o_ref):
    o_ref[...] = lax.rsqrt(d_ref[0] + d_ref[1] + 1.0)


def _dinv(degp, blk=2000):
    n = degp.shape[1]
    return pl.pallas_call(
        _dinv_body,
        grid=(n // blk,),
        in_specs=[pl.BlockSpec((2, blk, 1), lambda i: (0, i, 0))],
        out_specs=pl.BlockSpec((blk, 1), lambda i: (i, 0)),
        out_shape=jax.ShapeDtypeStruct((n, 1), jnp.float32),
    )(degp)


_NC = 2
_NS = 16
_NW = _NC * _NS
_EPW = E // _NW
_K = 125
_CH = _EPW // _K
_RPT = N // _NS
_DEGW = 16
_PW = 128
_NPH = H // _PW
_NP0 = 5120
_NP1 = 4992
_R0 = _NP0 // _NS
_R1 = _NP1 // _NS

_sc_mesh = plsc.VectorSubcoreMesh(core_axis_name="c", subcore_axis_name="s")


def _deg_sc(d0_r, d1_r, zeros_agg, ones_rows):
    @functools.partial(
        pl.kernel,
        out_type=[jax.ShapeDtypeStruct((_NW, r, _PW), jnp.float32)
                  for r in (_R0, _R1)],
        mesh=_sc_mesh,
        scratch_types=[
            pltpu.VMEM((_CH, _K), jnp.int32),
            pltpu.VMEM((_CH, _K), jnp.int32),
            pltpu.VMEM((_K, _PW), jnp.float32),
            pltpu.VMEM_SHARED((_NP0 + 8, _PW), jnp.float32),
        ],
    )
    def k(d0_hbm, d1_hbm, z_hbm, ones_hbm, out0_hbm, out1_hbm,
          didx0, didx1, ones_v, acc):
        c = lax.axis_index("c")
        s = lax.axis_index("s")
        w = c * _NS + s
        pltpu.sync_copy(d0_hbm.at[w], didx0)
        pltpu.sync_copy(d1_hbm.at[w], didx1)
        pltpu.sync_copy(ones_hbm, ones_v)
        for didx, rpt, out in ((didx0, _R0, out0_hbm), (didx1, _R1, out1_hbm)):
            pltpu.sync_copy(z_hbm.at[pl.ds(0, rpt)],
                            acc.at[pl.ds(s * rpt, rpt)])
            plsc.subcore_barrier()

            def body(j, carry):
                pltpu.sync_copy(ones_v, acc.at[didx.at[j]], add=True)
                return carry

            lax.fori_loop(0, _CH, body, 0)
            plsc.subcore_barrier()
            pltpu.sync_copy(acc.at[pl.ds(s * rpt, rpt)], out.at[w])
            plsc.subcore_barrier()

    return k(d0_r, d1_r, zeros_agg, ones_rows)


def _agg_sc(src_r, d0_r, d1_r, xs_parts, zeros_agg):
    @functools.partial(
        pl.kernel,
        out_type=[jax.ShapeDtypeStruct((_NW, r, _PW), jnp.float32)
                  for _ in range(_NPH) for r in (_R0, _R1)],
        mesh=_sc_mesh,
        scratch_types=[
            pltpu.VMEM((_CH, _K), jnp.int32),
            pltpu.VMEM((_CH, _K), jnp.int32),
            pltpu.VMEM((_CH, _K), jnp.int32),
            pltpu.VMEM((_K, _PW), jnp.float32),
            pltpu.VMEM((_K, _PW), jnp.float32),
            pltpu.VMEM_SHARED((_NP0 + 8, _PW), jnp.float32),
            pltpu.SemaphoreType.DMA,
            pltpu.SemaphoreType.DMA,
        ],
    )
    def k(src_hbm, d0_hbm, d1_hbm, *rest):
        tabs = rest[:_NPH]
        z_hbm = rest[_NPH]
        outs = rest[_NPH + 1:3 * _NPH + 1]
        sidx, didx0, didx1, buf_a, buf_b, acc, sem_a, sem_b = \
            rest[3 * _NPH + 1:]
        c = lax.axis_index("c")
        s = lax.axis_index("s")
        w = c * _NS + s
        pltpu.sync_copy(src_hbm.at[w], sidx)
        pltpu.sync_copy(d0_hbm.at[w], didx0)
        pltpu.sync_copy(d1_hbm.at[w], didx1)
        oi = 0
        for tab in tabs:
            for didx, rpt in ((didx0, _R0), (didx1, _R1)):
                out = outs[oi]
                oi += 1
                pltpu.sync_copy(z_hbm.at[pl.ds(0, rpt)],
                                acc.at[pl.ds(s * rpt, rpt)])
                plsc.subcore_barrier()
                pltpu.async_copy(tab.at[sidx.at[0]], buf_a, sem_a)

                def body(jj, carry):
                    for b, buf, sem, obuf, osem in (
                            (0, buf_a, sem_a, buf_b, sem_b),
                            (1, buf_b, sem_b, buf_a, sem_a)):
                        j = 2 * jj + b
                        pltpu.make_async_copy(
                            tab.at[sidx.at[j]], buf, sem).wait()

                        @pl.when(j + 1 < _CH)
                        def _next():
                            pltpu.async_copy(
                                tab.at[sidx.at[j + 1]], obuf, osem)

                        pltpu.sync_copy(buf, acc.at[didx.at[j]], add=True)
                    return carry

                lax.fori_loop(0, _CH // 2, body, 0)
                plsc.subcore_barrier()
                pltpu.sync_copy(acc.at[pl.ds(s * rpt, rpt)], out.at[w])
                plsc.subcore_barrier()

    return k(src_r, d0_r, d1_r, *xs_parts, zeros_agg)


def _deg_partials(d0_r, d1_r, zeros_agg, ones_rows):
    o0, o1 = _deg_sc(d0_r, d1_r, zeros_agg, ones_rows)
    p0 = o0.reshape(_NC, _NP0, _PW)
    p1 = o1.reshape(_NC, _NP1, _PW)
    return jnp.concatenate([p0, p1], axis=1)[:, :N, :1]


def _clamp_body(d_ref, o0_ref, o1_ref):
    d = d_ref[...]
    o0_ref[...] = jnp.where(d < _NP0, d, _NP0)
    o1_ref[...] = jnp.where(d >= _NP0, d - _NP0, _NP1)


def _clamp_idx(dst):
    d = dst.reshape(E // 128, 128)
    o0, o1 = pl.pallas_call(
        _clamp_body,
        out_shape=[jax.ShapeDtypeStruct((E // 128, 128), jnp.int32)] * 2,
    )(d)
    return o0.reshape(_NW, _CH, _K), o1.reshape(_NW, _CH, _K)


def _agg_partials(xs, src_r, d0_r, d1_r, zeros_agg):
    parts = [xs[:, p * _PW:(p + 1) * _PW] for p in range(_NPH)]
    outs = _agg_sc(src_r, d0_r, d1_r, parts, zeros_agg)
    halves = []
    for h in range(_NPH):
        p0 = outs[2 * h].reshape(_NC, _NP0, _PW)
        p1 = outs[2 * h + 1].reshape(_NC, _NP1, _PW)
        halves.append(jnp.concatenate([p0, p1], axis=1)[:, :N])
    aggA = jnp.concatenate([hv[0] for hv in halves], axis=1)
    aggB = jnp.concatenate([hv[1] for hv in halves], axis=1)
    return aggA, aggB


def kernel(x, edge_index, h0, c0, W1, b1, W2, b2, W_ih, W_hh, b_ih, b_hh,
           W_fc, b_fc):
    src_r = edge_index[0].reshape(_NW, _CH, _K)
    ones_rows = jnp.ones((_K, _PW), jnp.float32)
    zeros_agg = jnp.zeros((_R0, _PW), jnp.float32)

    d0_r, d1_r = _clamp_idx(edge_index[1])
    degp = _deg_partials(d0_r, d1_r, zeros_agg, ones_rows)
    dinv = _dinv(degp)

    xs1 = _mm_scale(x, W1, dinv)
    a1A, a1B = _agg_partials(xs1, src_r, d0_r, d1_r, zeros_agg)
    xs2 = _cmb_mm(a1A, a1B, xs1, dinv, b1[None, :], W2, b1[None, :],
                  scale_out=True)
    a2A, a2B = _agg_partials(xs2, src_r, d0_r, d1_r, zeros_agg)
    xi = _cmb_mm(a2A, a2B, xs2, dinv, b2[None, :], W_ih.T,
                 b_ih[None, :], scale_out=False)

    logp, hT, cT = _lstm_head(xi, h0[0], c0[0], W_hh.T, b_hh[None, :],
                              W_fc, b_fc[None, :])
    return logp[None, :, :], hT[None, :, :], cT[None, :, :]


# speedup vs baseline: 11.0013x; 1.0943x over previous
import functools

import jax
import jax.numpy as jnp
from jax import lax
from jax.experimental import pallas as pl
from jax.experimental.pallas import tpu as pltpu
from jax.experimental.pallas import tpu_sc as plsc

N = 10000
E = 320000
IN = 128
H = 256
LH = 256
OUT = 128


def _mm_scale_body(a_ref, w_ref, dinv_ref, o_ref):
    acc = jnp.dot(a_ref[...], w_ref[...], preferred_element_type=jnp.float32)
    o_ref[...] = acc * dinv_ref[...]


def _mm_scale(a, w, dinv, blk=2000):
    n, kin = a.shape
    kout = w.shape[1]
    grid = n // blk
    return pl.pallas_call(
        _mm_scale_body,
        grid=(grid,),
        in_specs=[
            pl.BlockSpec((blk, kin), lambda i: (i, 0)),
            pl.BlockSpec((kin, kout), lambda i: (0, 0)),
            pl.BlockSpec((blk, 1), lambda i: (i, 0)),
        ],
        out_specs=pl.BlockSpec((blk, kout), lambda i: (i, 0)),
        out_shape=jax.ShapeDtypeStruct((n, kout), jnp.float32),
    )(a, w, dinv)


def _cmb_mm_body(scale_out, aggA_ref, aggB_ref, xs_ref, dinv_ref, b_ref,
                 w_ref, b2_ref, o_ref):
    h = (aggA_ref[...] + aggB_ref[...] + xs_ref[...]) * dinv_ref[...] + b_ref[...]
    h = jnp.maximum(h, 0.0)
    acc = jnp.dot(h, w_ref[...], preferred_element_type=jnp.float32)
    if scale_out:
        o_ref[...] = acc * dinv_ref[...]
    else:
        o_ref[...] = acc + b2_ref[...]


def _cmb_mm(aggA, aggB, xs, dinv, b, w, b2, scale_out, blk=1000):
    n, kin = xs.shape
    kout = w.shape[1]
    grid = n // blk
    return pl.pallas_call(
        functools.partial(_cmb_mm_body, scale_out),
        grid=(grid,),
        in_specs=[
            pl.BlockSpec((blk, kin), lambda i: (i, 0)),
            pl.BlockSpec((blk, kin), lambda i: (i, 0)),
            pl.BlockSpec((blk, kin), lambda i: (i, 0)),
            pl.BlockSpec((blk, 1), lambda i: (i, 0)),
            pl.BlockSpec((1, kin), lambda i: (0, 0)),
            pl.BlockSpec((kin, kout), lambda i: (0, 0)),
            pl.BlockSpec((1, kout), lambda i: (0, 0)),
        ],
        out_specs=pl.BlockSpec((blk, kout), lambda i: (i, 0)),
        out_shape=jax.ShapeDtypeStruct((n, kout), jnp.float32),
    )(aggA, aggB, xs, dinv, b, w, b2)


def _lstm_body(tb, xi_ref, h0_ref, c0_ref, whh_ref, bhh_ref, wfc_ref, bfc_ref,
               logp_ref, hT_ref, cT_ref, h_s, c_s, hs_s):
    @pl.when(pl.program_id(0) == 0)
    def _init():
        h_s[...] = h0_ref[...]
        c_s[...] = c0_ref[...]

    def step(t, carry):
        h, c = carry
        g = xi_ref[pl.ds(t, 1), :] + jnp.dot(
            h.astype(jnp.bfloat16), whh_ref[...],
            preferred_element_type=jnp.float32) + bhh_ref[...]
        i = jax.nn.sigmoid(g[:, 0:LH])
        f = jax.nn.sigmoid(g[:, LH:2 * LH])
        gg = jnp.tanh(g[:, 2 * LH:3 * LH])
        o = jax.nn.sigmoid(g[:, 3 * LH:4 * LH])
        c = f * c + i * gg
        h = o * jnp.tanh(c)
        hs_s[pl.ds(t, 1), :] = h
        return (h, c)

    h, c = lax.fori_loop(0, tb, step, (h_s[...], c_s[...]), unroll=4)
    h_s[...] = h
    c_s[...] = c
    out = jnp.dot(hs_s[...], wfc_ref[...], preferred_element_type=jnp.float32) \
        + bfc_ref[...]
    m = jnp.max(out, axis=-1, keepdims=True)
    lse = jnp.log(jnp.sum(jnp.exp(out - m), axis=-1, keepdims=True))
    logp_ref[...] = out - m - lse
    hT_ref[...] = h
    cT_ref[...] = c


def _lstm_head(xi, h0, c0, whh_t, bhh, wfc, bfc, tb=400):
    n = xi.shape[0]
    grid = n // tb
    return pl.pallas_call(
        functools.partial(_lstm_body, tb),
        grid=(grid,),
        in_specs=[
            pl.BlockSpec((tb, 4 * LH), lambda i: (i, 0)),
            pl.BlockSpec((1, LH), lambda i: (0, 0)),
            pl.BlockSpec((1, LH), lambda i: (0, 0)),
            pl.BlockSpec((LH, 4 * LH), lambda i: (0, 0)),
            pl.BlockSpec((1, 4 * LH), lambda i: (0, 0)),
            pl.BlockSpec((LH, OUT), lambda i: (0, 0)),
            pl.BlockSpec((1, OUT), lambda i: (0, 0)),
        ],
        out_specs=[
            pl.BlockSpec((tb, OUT), lambda i: (i, 0)),
            pl.BlockSpec((1, LH), lambda i: (0, 0)),
            pl.BlockSpec((1, LH), lambda i: (0, 0)),
        ],
        out_shape=[
            jax.ShapeDtypeStruct((n, OUT), jnp.float32),
            jax.ShapeDtypeStruct((1, LH), jnp.float32),
            jax.ShapeDtypeStruct((1, LH), jnp.float32),
        ],
        scratch_shapes=[
            pltpu.VMEM((1, LH), jnp.float32),
            pltpu.VMEM((1, LH), jnp.float32),
            pltpu.VMEM((tb, LH), jnp.float32),
        ],
    )(xi, h0, c0, whh_t, bhh, wfc, bfc)


def _dinv_body(d_ref, o_ref):
    o_ref[...] = lax.rsqrt(d_ref[0] + d_ref[1] + 1.0)


def _dinv(degp, blk=2000):
    n = degp.shape[1]
    return pl.pallas_call(
        _dinv_body,
        grid=(n // blk,),
        in_specs=[pl.BlockSpec((2, blk, 1), lambda i: (0, i, 0))],
        out_specs=pl.BlockSpec((blk, 1), lambda i: (i, 0)),
        out_shape=jax.ShapeDtypeStruct((n, 1), jnp.float32),
    )(degp)


_NC = 2
_NS = 16
_NW = _NC * _NS
_EPW = E // _NW
_K = 125
_CH = _EPW // _K
_RPT = N // _NS
_DEGW = 16
_PW = 128
_NPH = H // _PW
_NP0 = 5120
_NP1 = 4992
_R0 = _NP0 // _NS
_R1 = _NP1 // _NS

_sc_mesh = plsc.VectorSubcoreMesh(core_axis_name="c", subcore_axis_name="s")


def _deg_sc(d0_r, d1_r, zeros_agg, ones_rows):
    @functools.partial(
        pl.kernel,
        out_type=[jax.ShapeDtypeStruct((_NW, r, _PW), jnp.float32)
                  for r in (_R0, _R1)],
        mesh=_sc_mesh,
        scratch_types=[
            pltpu.VMEM((_CH, _K), jnp.int32),
            pltpu.VMEM((_CH, _K), jnp.int32),
            pltpu.VMEM((_K, _PW), jnp.float32),
            pltpu.VMEM_SHARED((_NP0 + 8, _PW), jnp.float32),
        ],
    )
    def k(d0_hbm, d1_hbm, z_hbm, ones_hbm, out0_hbm, out1_hbm,
          didx0, didx1, ones_v, acc):
        c = lax.axis_index("c")
        s = lax.axis_index("s")
        w = c * _NS + s
        pltpu.sync_copy(d0_hbm.at[w], didx0)
        pltpu.sync_copy(d1_hbm.at[w], didx1)
        pltpu.sync_copy(ones_hbm, ones_v)
        for didx, rpt, out in ((didx0, _R0, out0_hbm), (didx1, _R1, out1_hbm)):
            pltpu.sync_copy(z_hbm.at[pl.ds(0, rpt)],
                            acc.at[pl.ds(s * rpt, rpt)])
            plsc.subcore_barrier()

            def body(j, carry):
                pltpu.sync_copy(ones_v, acc.at[didx.at[j]], add=True)
                return carry

            lax.fori_loop(0, _CH, body, 0)
            plsc.subcore_barrier()
            pltpu.sync_copy(acc.at[pl.ds(s * rpt, rpt)], out.at[w])
            plsc.subcore_barrier()

    return k(d0_r, d1_r, zeros_agg, ones_rows)


def _agg_sc(src_r, d0_r, d1_r, xs_parts, zeros_agg):
    @functools.partial(
        pl.kernel,
        out_type=[jax.ShapeDtypeStruct((_NW, r, _PW), jnp.float32)
                  for _ in range(_NPH) for r in (_R0, _R1)],
        mesh=_sc_mesh,
        scratch_types=[
            pltpu.VMEM((_CH, _K), jnp.int32),
            pltpu.VMEM((_CH, _K), jnp.int32),
            pltpu.VMEM((_CH, _K), jnp.int32),
            pltpu.VMEM((_K, _PW), jnp.float32),
            pltpu.VMEM((_K, _PW), jnp.float32),
            pltpu.VMEM_SHARED((_NP0 + 8, _PW), jnp.float32),
            pltpu.SemaphoreType.DMA,
            pltpu.SemaphoreType.DMA,
        ],
    )
    def k(src_hbm, d0_hbm, d1_hbm, *rest):
        tabs = rest[:_NPH]
        z_hbm = rest[_NPH]
        outs = rest[_NPH + 1:3 * _NPH + 1]
        sidx, didx0, didx1, buf_a, buf_b, acc, sem_a, sem_b = \
            rest[3 * _NPH + 1:]
        c = lax.axis_index("c")
        s = lax.axis_index("s")
        w = c * _NS + s
        pltpu.sync_copy(src_hbm.at[w], sidx)
        pltpu.sync_copy(d0_hbm.at[w], didx0)
        pltpu.sync_copy(d1_hbm.at[w], didx1)
        oi = 0
        for tab in tabs:
            for didx, rpt in ((didx0, _R0), (didx1, _R1)):
                out = outs[oi]
                oi += 1
                pltpu.sync_copy(z_hbm.at[pl.ds(0, rpt)],
                                acc.at[pl.ds(s * rpt, rpt)])
                plsc.subcore_barrier()
                pltpu.async_copy(tab.at[sidx.at[0]], buf_a, sem_a)

                def body(jj, carry):
                    for b, buf, sem, obuf, osem in (
                            (0, buf_a, sem_a, buf_b, sem_b),
                            (1, buf_b, sem_b, buf_a, sem_a)):
                        j = 2 * jj + b
                        pltpu.make_async_copy(
                            tab.at[sidx.at[j]], buf, sem).wait()

                        @pl.when(j + 1 < _CH)
                        def _next():
                            pltpu.async_copy(
                                tab.at[sidx.at[j + 1]], obuf, osem)

                        pltpu.sync_copy(buf, acc.at[didx.at[j]], add=True)
                    return carry

                lax.fori_loop(0, _CH // 2, body, 0)
                plsc.subcore_barrier()
                pltpu.sync_copy(acc.at[pl.ds(s * rpt, rpt)], out.at[w])
                plsc.subcore_barrier()

    return k(src_r, d0_r, d1_r, *xs_parts, zeros_agg)


def _deg_partials(d0_r, d1_r, zeros_agg, ones_rows):
    o0, o1 = _deg_sc(d0_r, d1_r, zeros_agg, ones_rows)
    p0 = o0.reshape(_NC, _NP0, _PW)
    p1 = o1.reshape(_NC, _NP1, _PW)
    return jnp.concatenate([p0, p1], axis=1)[:, :N, :1]


def _clamp_body(d_ref, o0_ref, o1_ref):
    d = d_ref[...]
    o0_ref[...] = jnp.where(d < _NP0, d, _NP0)
    o1_ref[...] = jnp.where(d >= _NP0, d - _NP0, _NP1)


def _clamp_idx(dst):
    d = dst.reshape(E // 128, 128)
    o0, o1 = pl.pallas_call(
        _clamp_body,
        out_shape=[jax.ShapeDtypeStruct((E // 128, 128), jnp.int32)] * 2,
    )(d)
    return o0.reshape(_NW, _CH, _K), o1.reshape(_NW, _CH, _K)


def _agg_partials(xs, src_r, d0_r, d1_r, zeros_agg):
    parts = [xs[:, p * _PW:(p + 1) * _PW] for p in range(_NPH)]
    outs = _agg_sc(src_r, d0_r, d1_r, parts, zeros_agg)
    halves = []
    for h in range(_NPH):
        p0 = outs[2 * h].reshape(_NC, _NP0, _PW)
        p1 = outs[2 * h + 1].reshape(_NC, _NP1, _PW)
        halves.append(jnp.concatenate([p0, p1], axis=1)[:, :N])
    aggA = jnp.concatenate([hv[0] for hv in halves], axis=1)
    aggB = jnp.concatenate([hv[1] for hv in halves], axis=1)
    return aggA, aggB


def kernel(x, edge_index, h0, c0, W1, b1, W2, b2, W_ih, W_hh, b_ih, b_hh,
           W_fc, b_fc):
    src_r = edge_index[0].reshape(_NW, _CH, _K)
    ones_rows = jnp.ones((_K, _PW), jnp.float32)
    zeros_agg = jnp.zeros((_R0, _PW), jnp.float32)

    d0_r, d1_r = _clamp_idx(edge_index[1])
    degp = _deg_partials(d0_r, d1_r, zeros_agg, ones_rows)
    dinv = _dinv(degp)

    xs1 = _mm_scale(x, W1, dinv)
    a1A, a1B = _agg_partials(xs1, src_r, d0_r, d1_r, zeros_agg)
    xs2 = _cmb_mm(a1A, a1B, xs1, dinv, b1[None, :], W2, b1[None, :],
                  scale_out=True)
    a2A, a2B = _agg_partials(xs2, src_r, d0_r, d1_r, zeros_agg)
    xi = _cmb_mm(a2A, a2B, xs2, dinv, b2[None, :], W_ih.T,
                 b_ih[None, :], scale_out=False)

    logp, hT, cT = _lstm_head(xi, h0[0], c0[0],
                              W_hh.T.astype(jnp.bfloat16), b_hh[None, :],
                              W_fc, b_fc[None, :])
    return logp[None, :, :], hT[None, :, :], cT[None, :, :]


# speedup vs baseline: 11.4612x; 1.0418x over previous
import functools

import jax
import jax.numpy as jnp
from jax import lax
from jax.experimental import pallas as pl
from jax.experimental.pallas import tpu as pltpu
from jax.experimental.pallas import tpu_sc as plsc

N = 10000
E = 320000
IN = 128
H = 256
LH = 256
OUT = 128


def _mm_scale_body(a_ref, w_ref, dinv_ref, o_ref):
    acc = jnp.dot(a_ref[...], w_ref[...], preferred_element_type=jnp.float32)
    o_ref[...] = acc * dinv_ref[...]


def _mm_scale(a, w, dinv, blk=2000):
    n, kin = a.shape
    kout = w.shape[1]
    grid = n // blk
    return pl.pallas_call(
        _mm_scale_body,
        grid=(grid,),
        in_specs=[
            pl.BlockSpec((blk, kin), lambda i: (i, 0)),
            pl.BlockSpec((kin, kout), lambda i: (0, 0)),
            pl.BlockSpec((blk, 1), lambda i: (i, 0)),
        ],
        out_specs=pl.BlockSpec((blk, kout), lambda i: (i, 0)),
        out_shape=jax.ShapeDtypeStruct((n, kout), jnp.float32),
    )(a, w, dinv)


def _cmb_mm_body(scale_out, aggA_ref, aggB_ref, xs_ref, dinv_ref, b_ref,
                 w_ref, b2_ref, o_ref):
    h = (aggA_ref[...] + aggB_ref[...] + xs_ref[...]) * dinv_ref[...] + b_ref[...]
    h = jnp.maximum(h, 0.0)
    acc = jnp.dot(h, w_ref[...], preferred_element_type=jnp.float32)
    if scale_out:
        o_ref[...] = acc * dinv_ref[...]
    else:
        o_ref[...] = acc + b2_ref[...]


def _cmb_mm(aggA, aggB, xs, dinv, b, w, b2, scale_out, blk=1000):
    n, kin = xs.shape
    kout = w.shape[1]
    grid = n // blk
    return pl.pallas_call(
        functools.partial(_cmb_mm_body, scale_out),
        grid=(grid,),
        in_specs=[
            pl.BlockSpec((blk, kin), lambda i: (i, 0)),
            pl.BlockSpec((blk, kin), lambda i: (i, 0)),
            pl.BlockSpec((blk, kin), lambda i: (i, 0)),
            pl.BlockSpec((blk, 1), lambda i: (i, 0)),
            pl.BlockSpec((1, kin), lambda i: (0, 0)),
            pl.BlockSpec((kin, kout), lambda i: (0, 0)),
            pl.BlockSpec((1, kout), lambda i: (0, 0)),
        ],
        out_specs=pl.BlockSpec((blk, kout), lambda i: (i, 0)),
        out_shape=jax.ShapeDtypeStruct((n, kout), jnp.float32),
    )(aggA, aggB, xs, dinv, b, w, b2)


def _lstm_body(tb, xi_ref, h0_ref, c0_ref, whh_ref, bhh_ref, wfc_ref, bfc_ref,
               logp_ref, hT_ref, cT_ref, h_s, c_s, hs_s):
    @pl.when(pl.program_id(0) == 0)
    def _init():
        h_s[...] = jnp.broadcast_to(h0_ref[...], (8, LH))
        c_s[...] = jnp.broadcast_to(c0_ref[...], (8, LH))

    def step(t, carry):
        h, c = carry
        g = xi_ref[pl.ds(t, 1), :] + jnp.dot(
            h.astype(jnp.bfloat16), whh_ref[...],
            preferred_element_type=jnp.float32) + bhh_ref[...]
        i = jax.nn.sigmoid(g[:, 0:LH])
        f = jax.nn.sigmoid(g[:, LH:2 * LH])
        gg = jnp.tanh(g[:, 2 * LH:3 * LH])
        o = jax.nn.sigmoid(g[:, 3 * LH:4 * LH])
        c = f * c + i * gg
        h = o * jnp.tanh(c)
        hs_s[pl.ds(t, 1), :] = h[0:1]
        return (h, c)

    h, c = lax.fori_loop(0, tb, step, (h_s[...], c_s[...]), unroll=4)
    h_s[...] = h
    c_s[...] = c
    out = jnp.dot(hs_s[...], wfc_ref[...], preferred_element_type=jnp.float32) \
        + bfc_ref[...]
    m = jnp.max(out, axis=-1, keepdims=True)
    lse = jnp.log(jnp.sum(jnp.exp(out - m), axis=-1, keepdims=True))
    logp_ref[...] = out - m - lse
    hT_ref[...] = h[0:1]
    cT_ref[...] = c[0:1]


def _lstm_head(xi, h0, c0, whh_t, bhh, wfc, bfc, tb=400):
    n = xi.shape[0]
    grid = n // tb
    return pl.pallas_call(
        functools.partial(_lstm_body, tb),
        grid=(grid,),
        in_specs=[
            pl.BlockSpec((tb, 4 * LH), lambda i: (i, 0)),
            pl.BlockSpec((1, LH), lambda i: (0, 0)),
            pl.BlockSpec((1, LH), lambda i: (0, 0)),
            pl.BlockSpec((LH, 4 * LH), lambda i: (0, 0)),
            pl.BlockSpec((1, 4 * LH), lambda i: (0, 0)),
            pl.BlockSpec((LH, OUT), lambda i: (0, 0)),
            pl.BlockSpec((1, OUT), lambda i: (0, 0)),
        ],
        out_specs=[
            pl.BlockSpec((tb, OUT), lambda i: (i, 0)),
            pl.BlockSpec((1, LH), lambda i: (0, 0)),
            pl.BlockSpec((1, LH), lambda i: (0, 0)),
        ],
        out_shape=[
            jax.ShapeDtypeStruct((n, OUT), jnp.float32),
            jax.ShapeDtypeStruct((1, LH), jnp.float32),
            jax.ShapeDtypeStruct((1, LH), jnp.float32),
        ],
        scratch_shapes=[
            pltpu.VMEM((8, LH), jnp.float32),
            pltpu.VMEM((8, LH), jnp.float32),
            pltpu.VMEM((tb, LH), jnp.float32),
        ],
    )(xi, h0, c0, whh_t, bhh, wfc, bfc)


def _dinv_body(d_ref, o_ref):
    o_ref[...] = lax.rsqrt(d_ref[0] + d_ref[1] + 1.0)


def _dinv(degp, blk=2000):
    n = degp.shape[1]
    return pl.pallas_call(
        _dinv_body,
        grid=(n // blk,),
        in_specs=[pl.BlockSpec((2, blk, 1), lambda i: (0, i, 0))],
        out_specs=pl.BlockSpec((blk, 1), lambda i: (i, 0)),
        out_shape=jax.ShapeDtypeStruct((n, 1), jnp.float32),
    )(degp)


_NC = 2
_NS = 16
_NW = _NC * _NS
_EPW = E // _NW
_K = 125
_CH = _EPW // _K
_RPT = N // _NS
_DEGW = 16
_PW = 128
_NPH = H // _PW
_NP0 = 5120
_NP1 = 4992
_R0 = _NP0 // _NS
_R1 = _NP1 // _NS

_sc_mesh = plsc.VectorSubcoreMesh(core_axis_name="c", subcore_axis_name="s")


def _deg_sc(d0_r, d1_r, zeros_agg, ones_rows):
    @functools.partial(
        pl.kernel,
        out_type=[jax.ShapeDtypeStruct((_NW, r, _PW), jnp.float32)
                  for r in (_R0, _R1)],
        mesh=_sc_mesh,
        scratch_types=[
            pltpu.VMEM((_CH, _K), jnp.int32),
            pltpu.VMEM((_CH, _K), jnp.int32),
            pltpu.VMEM((_K, _PW), jnp.float32),
            pltpu.VMEM_SHARED((_NP0 + 8, _PW), jnp.float32),
        ],
    )
    def k(d0_hbm, d1_hbm, z_hbm, ones_hbm, out0_hbm, out1_hbm,
          didx0, didx1, ones_v, acc):
        c = lax.axis_index("c")
        s = lax.axis_index("s")
        w = c * _NS + s
        pltpu.sync_copy(d0_hbm.at[w], didx0)
        pltpu.sync_copy(d1_hbm.at[w], didx1)
        pltpu.sync_copy(ones_hbm, ones_v)
        for didx, rpt, out in ((didx0, _R0, out0_hbm), (didx1, _R1, out1_hbm)):
            pltpu.sync_copy(z_hbm.at[pl.ds(0, rpt)],
                            acc.at[pl.ds(s * rpt, rpt)])
            plsc.subcore_barrier()

            def body(j, carry):
                pltpu.sync_copy(ones_v, acc.at[didx.at[j]], add=True)
                return carry

            lax.fori_loop(0, _CH, body, 0)
            plsc.subcore_barrier()
            pltpu.sync_copy(acc.at[pl.ds(s * rpt, rpt)], out.at[w])
            plsc.subcore_barrier()

    return k(d0_r, d1_r, zeros_agg, ones_rows)


def _agg_sc(src_r, d0_r, d1_r, xs_parts, zeros_agg):
    @functools.partial(
        pl.kernel,
        out_type=[jax.ShapeDtypeStruct((_NW, r, _PW), jnp.float32)
                  for _ in range(_NPH) for r in (_R0, _R1)],
        mesh=_sc_mesh,
        scratch_types=[
            pltpu.VMEM((_CH, _K), jnp.int32),
            pltpu.VMEM((_CH, _K), jnp.int32),
            pltpu.VMEM((_CH, _K), jnp.int32),
            pltpu.VMEM((_K, _PW), jnp.float32),
            pltpu.VMEM((_K, _PW), jnp.float32),
            pltpu.VMEM_SHARED((_NP0 + 8, _PW), jnp.float32),
            pltpu.SemaphoreType.DMA,
            pltpu.SemaphoreType.DMA,
            pltpu.SemaphoreType.DMA,
            pltpu.SemaphoreType.DMA,
        ],
    )
    def k(src_hbm, d0_hbm, d1_hbm, *rest):
        tabs = rest[:_NPH]
        z_hbm = rest[_NPH]
        outs = rest[_NPH + 1:3 * _NPH + 1]
        (sidx, didx0, didx1, buf_a, buf_b, acc,
         sem_a, sem_b, ssem_a, ssem_b) = rest[3 * _NPH + 1:]
        c = lax.axis_index("c")
        s = lax.axis_index("s")
        w = c * _NS + s
        pltpu.sync_copy(src_hbm.at[w], sidx)
        pltpu.sync_copy(d0_hbm.at[w], didx0)
        pltpu.sync_copy(d1_hbm.at[w], didx1)
        oi = 0
        for tab in tabs:
            for didx, rpt in ((didx0, _R0), (didx1, _R1)):
                out = outs[oi]
                oi += 1
                pltpu.sync_copy(z_hbm.at[pl.ds(0, rpt)],
                                acc.at[pl.ds(s * rpt, rpt)])
                plsc.subcore_barrier()
                pltpu.async_copy(tab.at[sidx.at[0]], buf_a, sem_a)

                def body(jj, carry):
                    for b, buf, sem, ssem, obuf, osem, ossem in (
                            (0, buf_a, sem_a, ssem_a, buf_b, sem_b, ssem_b),
                            (1, buf_b, sem_b, ssem_b, buf_a, sem_a, ssem_a)):
                        j = 2 * jj + b
                        pltpu.make_async_copy(
                            tab.at[sidx.at[j]], buf, sem).wait()
                        pltpu.async_copy(buf, acc.at[didx.at[j]], ssem,
                                         add=True)

                        @pl.when(j >= 1)
                        def _wprev():
                            pltpu.make_async_copy(
                                obuf, acc.at[didx.at[j - 1]], ossem).wait()

                        @pl.when(j + 1 < _CH)
                        def _next():
                            pltpu.async_copy(
                                tab.at[sidx.at[j + 1]], obuf, osem)
                    return carry

                lax.fori_loop(0, _CH // 2, body, 0)
                pltpu.make_async_copy(buf_b, acc.at[didx.at[_CH - 1]],
                                      ssem_b).wait()
                plsc.subcore_barrier()
                pltpu.sync_copy(acc.at[pl.ds(s * rpt, rpt)], out.at[w])
                plsc.subcore_barrier()

    return k(src_r, d0_r, d1_r, *xs_parts, zeros_agg)


def _deg_partials(d0_r, d1_r, zeros_agg, ones_rows):
    o0, o1 = _deg_sc(d0_r, d1_r, zeros_agg, ones_rows)
    p0 = o0.reshape(_NC, _NP0, _PW)
    p1 = o1.reshape(_NC, _NP1, _PW)
    return jnp.concatenate([p0, p1], axis=1)[:, :N, :1]


def _clamp_body(d_ref, o0_ref, o1_ref):
    d = d_ref[...]
    o0_ref[...] = jnp.where(d < _NP0, d, _NP0)
    o1_ref[...] = jnp.where(d >= _NP0, d - _NP0, _NP1)


def _clamp_idx(dst):
    d = dst.reshape(E // 128, 128)
    o0, o1 = pl.pallas_call(
        _clamp_body,
        out_shape=[jax.ShapeDtypeStruct((E // 128, 128), jnp.int32)] * 2,
    )(d)
    return o0.reshape(_NW, _CH, _K), o1.reshape(_NW, _CH, _K)


def _agg_partials(xs, src_r, d0_r, d1_r, zeros_agg):
    parts = [xs[:, p * _PW:(p + 1) * _PW] for p in range(_NPH)]
    outs = _agg_sc(src_r, d0_r, d1_r, parts, zeros_agg)
    halves = []
    for h in range(_NPH):
        p0 = outs[2 * h].reshape(_NC, _NP0, _PW)
        p1 = outs[2 * h + 1].reshape(_NC, _NP1, _PW)
        halves.append(jnp.concatenate([p0, p1], axis=1)[:, :N])
    aggA = jnp.concatenate([hv[0] for hv in halves], axis=1)
    aggB = jnp.concatenate([hv[1] for hv in halves], axis=1)
    return aggA, aggB


def kernel(x, edge_index, h0, c0, W1, b1, W2, b2, W_ih, W_hh, b_ih, b_hh,
           W_fc, b_fc):
    src_r = edge_index[0].reshape(_NW, _CH, _K)
    ones_rows = jnp.ones((_K, _PW), jnp.float32)
    zeros_agg = jnp.zeros((_R0, _PW), jnp.float32)

    d0_r, d1_r = _clamp_idx(edge_index[1])
    degp = _deg_partials(d0_r, d1_r, zeros_agg, ones_rows)
    dinv = _dinv(degp)

    xs1 = _mm_scale(x, W1, dinv)
    a1A, a1B = _agg_partials(xs1, src_r, d0_r, d1_r, zeros_agg)
    xs2 = _cmb_mm(a1A, a1B, xs1, dinv, b1[None, :], W2, b1[None, :],
                  scale_out=True)
    a2A, a2B = _agg_partials(xs2, src_r, d0_r, d1_r, zeros_agg)
    xi = _cmb_mm(a2A, a2B, xs2, dinv, b2[None, :], W_ih.T,
                 b_ih[None, :], scale_out=False)

    logp, hT, cT = _lstm_head(xi, h0[0], c0[0],
                              W_hh.T.astype(jnp.bfloat16), b_hh[None, :],
                              W_fc, b_fc[None, :])
    return logp[None, :, :], hT[None, :, :], cT[None, :, :]


# speedup vs baseline: 11.4916x; 1.0027x over previous
import functools

import jax
import jax.numpy as jnp
from jax import lax
from jax.experimental import pallas as pl
from jax.experimental.pallas import tpu as pltpu
from jax.experimental.pallas import tpu_sc as plsc

N = 10000
E = 320000
IN = 128
H = 256
LH = 256
OUT = 128


def _mm_scale_body(a_ref, w_ref, dinv_ref, o_ref):
    acc = jnp.dot(a_ref[...], w_ref[...], preferred_element_type=jnp.float32)
    o_ref[...] = acc * dinv_ref[...]


def _mm_scale(a, w, dinv, blk=2000):
    n, kin = a.shape
    kout = w.shape[1]
    grid = n // blk
    return pl.pallas_call(
        _mm_scale_body,
        grid=(grid,),
        in_specs=[
            pl.BlockSpec((blk, kin), lambda i: (i, 0)),
            pl.BlockSpec((kin, kout), lambda i: (0, 0)),
            pl.BlockSpec((blk, 1), lambda i: (i, 0)),
        ],
        out_specs=pl.BlockSpec((blk, kout), lambda i: (i, 0)),
        out_shape=jax.ShapeDtypeStruct((n, kout), jnp.float32),
    )(a, w, dinv)


def _cmb_mm_body(scale_out, aggA_ref, aggB_ref, xs_ref, dinv_ref, b_ref,
                 w_ref, b2_ref, o_ref):
    h = (aggA_ref[...] + aggB_ref[...] + xs_ref[...]) * dinv_ref[...] + b_ref[...]
    h = jnp.maximum(h, 0.0)
    acc = jnp.dot(h, w_ref[...], preferred_element_type=jnp.float32)
    if scale_out:
        o_ref[...] = acc * dinv_ref[...]
    else:
        o_ref[...] = acc + b2_ref[...]


def _cmb_mm(aggA, aggB, xs, dinv, b, w, b2, scale_out, blk=1000):
    n, kin = xs.shape
    kout = w.shape[1]
    grid = n // blk
    return pl.pallas_call(
        functools.partial(_cmb_mm_body, scale_out),
        grid=(grid,),
        in_specs=[
            pl.BlockSpec((blk, kin), lambda i: (i, 0)),
            pl.BlockSpec((blk, kin), lambda i: (i, 0)),
            pl.BlockSpec((blk, kin), lambda i: (i, 0)),
            pl.BlockSpec((blk, 1), lambda i: (i, 0)),
            pl.BlockSpec((1, kin), lambda i: (0, 0)),
            pl.BlockSpec((kin, kout), lambda i: (0, 0)),
            pl.BlockSpec((1, kout), lambda i: (0, 0)),
        ],
        out_specs=pl.BlockSpec((blk, kout), lambda i: (i, 0)),
        out_shape=jax.ShapeDtypeStruct((n, kout), jnp.float32),
    )(aggA, aggB, xs, dinv, b, w, b2)


def _lstm_body(tb, xi_ref, h0_ref, c0_ref, whh_ref, bhh_ref, wfc_ref, bfc_ref,
               logp_ref, hT_ref, cT_ref, h_s, c_s, hs_s):
    @pl.when(pl.program_id(0) == 0)
    def _init():
        h_s[...] = jnp.broadcast_to(h0_ref[...], (8, LH))
        c_s[...] = jnp.broadcast_to(c0_ref[...], (8, LH))

    def step(t, carry):
        h, c = carry
        g = xi_ref[pl.ds(t, 1), :] + jnp.dot(
            h.astype(jnp.bfloat16), whh_ref[...],
            preferred_element_type=jnp.float32) + bhh_ref[...]
        i = jax.nn.sigmoid(g[:, 0:LH])
        f = jax.nn.sigmoid(g[:, LH:2 * LH])
        gg = jnp.tanh(g[:, 2 * LH:3 * LH])
        o = jax.nn.sigmoid(g[:, 3 * LH:4 * LH])
        c = f * c + i * gg
        h = o * jnp.tanh(c)
        hs_s[pl.ds(t, 1), :] = h[0:1]
        return (h, c)

    h, c = lax.fori_loop(0, tb, step, (h_s[...], c_s[...]), unroll=8)
    h_s[...] = h
    c_s[...] = c
    out = jnp.dot(hs_s[...], wfc_ref[...], preferred_element_type=jnp.float32) \
        + bfc_ref[...]
    m = jnp.max(out, axis=-1, keepdims=True)
    lse = jnp.log(jnp.sum(jnp.exp(out - m), axis=-1, keepdims=True))
    logp_ref[...] = out - m - lse
    hT_ref[...] = h[0:1]
    cT_ref[...] = c[0:1]


def _lstm_head(xi, h0, c0, whh_t, bhh, wfc, bfc, tb=400):
    n = xi.shape[0]
    grid = n // tb
    return pl.pallas_call(
        functools.partial(_lstm_body, tb),
        grid=(grid,),
        in_specs=[
            pl.BlockSpec((tb, 4 * LH), lambda i: (i, 0)),
            pl.BlockSpec((1, LH), lambda i: (0, 0)),
            pl.BlockSpec((1, LH), lambda i: (0, 0)),
            pl.BlockSpec((LH, 4 * LH), lambda i: (0, 0)),
            pl.BlockSpec((1, 4 * LH), lambda i: (0, 0)),
            pl.BlockSpec((LH, OUT), lambda i: (0, 0)),
            pl.BlockSpec((1, OUT), lambda i: (0, 0)),
        ],
        out_specs=[
            pl.BlockSpec((tb, OUT), lambda i: (i, 0)),
            pl.BlockSpec((1, LH), lambda i: (0, 0)),
            pl.BlockSpec((1, LH), lambda i: (0, 0)),
        ],
        out_shape=[
            jax.ShapeDtypeStruct((n, OUT), jnp.float32),
            jax.ShapeDtypeStruct((1, LH), jnp.float32),
            jax.ShapeDtypeStruct((1, LH), jnp.float32),
        ],
        scratch_shapes=[
            pltpu.VMEM((8, LH), jnp.float32),
            pltpu.VMEM((8, LH), jnp.float32),
            pltpu.VMEM((tb, LH), jnp.float32),
        ],
    )(xi, h0, c0, whh_t, bhh, wfc, bfc)


def _dinv_body(d_ref, o_ref):
    o_ref[...] = lax.rsqrt(d_ref[0] + d_ref[1] + 1.0)


def _dinv(degp, blk=2000):
    n = degp.shape[1]
    return pl.pallas_call(
        _dinv_body,
        grid=(n // blk,),
        in_specs=[pl.BlockSpec((2, blk, 1), lambda i: (0, i, 0))],
        out_specs=pl.BlockSpec((blk, 1), lambda i: (i, 0)),
        out_shape=jax.ShapeDtypeStruct((n, 1), jnp.float32),
    )(degp)


_NC = 2
_NS = 16
_NW = _NC * _NS
_EPW = E // _NW
_K = 125
_CH = _EPW // _K
_RPT = N // _NS
_DEGW = 16
_PW = 128
_NPH = H // _PW
_NP0 = 5120
_NP1 = 4992
_R0 = _NP0 // _NS
_R1 = _NP1 // _NS

_sc_mesh = plsc.VectorSubcoreMesh(core_axis_name="c", subcore_axis_name="s")


def _deg_sc(d0_r, d1_r, zeros_agg, ones_rows):
    @functools.partial(
        pl.kernel,
        out_type=[jax.ShapeDtypeStruct((_NW, r, _PW), jnp.float32)
                  for r in (_R0, _R1)],
        mesh=_sc_mesh,
        scratch_types=[
            pltpu.VMEM((_CH, _K), jnp.int32),
            pltpu.VMEM((_CH, _K), jnp.int32),
            pltpu.VMEM((_K, _PW), jnp.float32),
            pltpu.VMEM_SHARED((_NP0 + 8, _PW), jnp.float32),
        ],
    )
    def k(d0_hbm, d1_hbm, z_hbm, ones_hbm, out0_hbm, out1_hbm,
          didx0, didx1, ones_v, acc):
        c = lax.axis_index("c")
        s = lax.axis_index("s")
        w = c * _NS + s
        pltpu.sync_copy(d0_hbm.at[w], didx0)
        pltpu.sync_copy(d1_hbm.at[w], didx1)
        pltpu.sync_copy(ones_hbm, ones_v)
        for didx, rpt, out in ((didx0, _R0, out0_hbm), (didx1, _R1, out1_hbm)):
            pltpu.sync_copy(z_hbm.at[pl.ds(0, rpt)],
                            acc.at[pl.ds(s * rpt, rpt)])
            plsc.subcore_barrier()

            def body(j, carry):
                pltpu.sync_copy(ones_v, acc.at[didx.at[j]], add=True)
                return carry

            lax.fori_loop(0, _CH, body, 0)
            plsc.subcore_barrier()
            pltpu.sync_copy(acc.at[pl.ds(s * rpt, rpt)], out.at[w])
            plsc.subcore_barrier()

    return k(d0_r, d1_r, zeros_agg, ones_rows)


def _agg_sc(src_r, d0_r, d1_r, xs_parts, zeros_agg, which=None):
    rpts = (_R0, _R1) if which is None else ((_R0, _R1)[which],)

    @functools.partial(
        pl.kernel,
        out_type=[jax.ShapeDtypeStruct((_NW, r, _PW), jnp.float32)
                  for _ in range(_NPH) for r in rpts],
        mesh=_sc_mesh,
        scratch_types=[
            pltpu.VMEM((_CH, _K), jnp.int32),
            pltpu.VMEM((_CH, _K), jnp.int32),
            pltpu.VMEM((_CH, _K), jnp.int32),
            pltpu.VMEM((_K, _PW), jnp.float32),
            pltpu.VMEM((_K, _PW), jnp.float32),
            pltpu.VMEM_SHARED((_NP0 + 8, _PW), jnp.float32),
            pltpu.SemaphoreType.DMA,
            pltpu.SemaphoreType.DMA,
            pltpu.SemaphoreType.DMA,
            pltpu.SemaphoreType.DMA,
        ],
    )
    def k(src_hbm, d0_hbm, d1_hbm, *rest):
        tabs = rest[:_NPH]
        z_hbm = rest[_NPH]
        n_out = _NPH * len(rpts)
        outs = rest[_NPH + 1:_NPH + 1 + n_out]
        (sidx, didx0, didx1, buf_a, buf_b, acc,
         sem_a, sem_b, ssem_a, ssem_b) = rest[_NPH + 1 + n_out:]
        c = lax.axis_index("c")
        s = lax.axis_index("s")
        w = c * _NS + s
        pltpu.sync_copy(src_hbm.at[w], sidx)
        if which in (None, 0):
            pltpu.sync_copy(d0_hbm.at[w], didx0)
        if which in (None, 1):
            pltpu.sync_copy(d1_hbm.at[w], didx1)
        node_passes = ((didx0, _R0), (didx1, _R1))
        if which is not None:
            node_passes = (node_passes[which],)
        oi = 0
        for tab in tabs:
            for didx, rpt in node_passes:
                out = outs[oi]
                oi += 1
                pltpu.sync_copy(z_hbm.at[pl.ds(0, rpt)],
                                acc.at[pl.ds(s * rpt, rpt)])
                plsc.subcore_barrier()
                pltpu.async_copy(tab.at[sidx.at[0]], buf_a, sem_a)

                def body(jj, carry):
                    for b, buf, sem, ssem, obuf, osem, ossem in (
                            (0, buf_a, sem_a, ssem_a, buf_b, sem_b, ssem_b),
                            (1, buf_b, sem_b, ssem_b, buf_a, sem_a, ssem_a)):
                        j = 2 * jj + b
                        pltpu.make_async_copy(
                            tab.at[sidx.at[j]], buf, sem).wait()
                        pltpu.async_copy(buf, acc.at[didx.at[j]], ssem,
                                         add=True)

                        @pl.when(j >= 1)
                        def _wprev():
                            pltpu.make_async_copy(
                                obuf, acc.at[didx.at[j - 1]], ossem).wait()

                        @pl.when(j + 1 < _CH)
                        def _next():
                            pltpu.async_copy(
                                tab.at[sidx.at[j + 1]], obuf, osem)
                    return carry

                lax.fori_loop(0, _CH // 2, body, 0)
                pltpu.make_async_copy(buf_b, acc.at[didx.at[_CH - 1]],
                                      ssem_b).wait()
                plsc.subcore_barrier()
                pltpu.sync_copy(acc.at[pl.ds(s * rpt, rpt)], out.at[w])
                plsc.subcore_barrier()

    return k(src_r, d0_r, d1_r, *xs_parts, zeros_agg)


def _deg_partials(d0_r, d1_r, zeros_agg, ones_rows):
    o0, o1 = _deg_sc(d0_r, d1_r, zeros_agg, ones_rows)
    p0 = o0.reshape(_NC, _NP0, _PW)
    p1 = o1.reshape(_NC, _NP1, _PW)
    return jnp.concatenate([p0, p1], axis=1)[:, :N, :1]


def _clamp_body(d_ref, o0_ref, o1_ref):
    d = d_ref[...]
    o0_ref[...] = jnp.where(d < _NP0, d, _NP0)
    o1_ref[...] = jnp.where(d >= _NP0, d - _NP0, _NP1)


def _clamp_idx(dst):
    d = dst.reshape(E // 128, 128)
    o0, o1 = pl.pallas_call(
        _clamp_body,
        out_shape=[jax.ShapeDtypeStruct((E // 128, 128), jnp.int32)] * 2,
    )(d)
    return o0.reshape(_NW, _CH, _K), o1.reshape(_NW, _CH, _K)


def _agg_partials(xs, src_r, d0_r, d1_r, zeros_agg):
    parts = [xs[:, p * _PW:(p + 1) * _PW] for p in range(_NPH)]
    outs = _agg_sc(src_r, d0_r, d1_r, parts, zeros_agg)
    halves = []
    for h in range(_NPH):
        p0 = outs[2 * h].reshape(_NC, _NP0, _PW)
        p1 = outs[2 * h + 1].reshape(_NC, _NP1, _PW)
        halves.append(jnp.concatenate([p0, p1], axis=1)[:, :N])
    aggA = jnp.concatenate([hv[0] for hv in halves], axis=1)
    aggB = jnp.concatenate([hv[1] for hv in halves], axis=1)
    return aggA, aggB


def kernel(x, edge_index, h0, c0, W1, b1, W2, b2, W_ih, W_hh, b_ih, b_hh,
           W_fc, b_fc):
    src_r = edge_index[0].reshape(_NW, _CH, _K)
    ones_rows = jnp.ones((_K, _PW), jnp.float32)
    zeros_agg = jnp.zeros((_R0, _PW), jnp.float32)

    d0_r, d1_r = _clamp_idx(edge_index[1])
    degp = _deg_partials(d0_r, d1_r, zeros_agg, ones_rows)
    dinv = _dinv(degp)

    xs1 = _mm_scale(x, W1, dinv)
    a1A, a1B = _agg_partials(xs1, src_r, d0_r, d1_r, zeros_agg)
    xs2 = _cmb_mm(a1A, a1B, xs1, dinv, b1[None, :], W2, b1[None, :],
                  scale_out=True)
    parts2 = [xs2[:, p * _PW:(p + 1) * _PW] for p in range(_NPH)]
    op0 = _agg_sc(src_r, d0_r, d1_r, parts2, zeros_agg, which=0)
    op1 = _agg_sc(src_r, d0_r, d1_r, parts2, zeros_agg, which=1)
    aA0 = jnp.concatenate(
        [o.reshape(_NC, _NP0, _PW)[0] for o in op0], axis=1)
    aB0 = jnp.concatenate(
        [o.reshape(_NC, _NP0, _PW)[1] for o in op0], axis=1)
    aA1 = jnp.concatenate(
        [o.reshape(_NC, _NP1, _PW)[0][:N - _NP0] for o in op1], axis=1)
    aB1 = jnp.concatenate(
        [o.reshape(_NC, _NP1, _PW)[1][:N - _NP0] for o in op1], axis=1)

    whh_bf = W_hh.T.astype(jnp.bfloat16)
    xi_a = _cmb_mm(aA0, aB0, xs2[:_NP0], dinv[:_NP0], b2[None, :], W_ih.T,
                   b_ih[None, :], scale_out=False, blk=640)
    xi_b = _cmb_mm(aA1, aB1, xs2[_NP0:], dinv[_NP0:], b2[None, :], W_ih.T,
                   b_ih[None, :], scale_out=False, blk=976)
    logp_a, hT1, cT1 = _lstm_head(xi_a, h0[0], c0[0], whh_bf, b_hh[None, :],
                                  W_fc, b_fc[None, :], tb=320)
    logp_b, hT, cT = _lstm_head(xi_b, hT1, cT1, whh_bf, b_hh[None, :],
                                W_fc, b_fc[None, :], tb=488)
    logp = jnp.concatenate([logp_a, logp_b], axis=0)
    return logp[None, :, :], hT[None, :, :], cT[None, :, :]
